# baseline (speedup 1.0000x reference)
import functools

import jax
import jax.numpy as jnp
from jax import lax
from jax.experimental import pallas as pl
from jax.experimental.pallas import tpu as pltpu
from jax.experimental.pallas import tpu_sc as plsc

VOCAB = 1000
VPAD = 1024
DIM = 128
BATCH = 1024
HIST = 50
VT = 16
NC, NS = 2, 16
NW = NC * NS
TILES_PER_W = (VPAD // VT) // NW
B_GROUPS = BATCH // 16
UNITS = TILES_PER_W * HIST
LAST_FULL_VT = VOCAB // VT - 1
PART_VT = VOCAB // VT
PART_ROWS = VOCAB % VT


def _table_body(head_ref, emb_ref, out_ref):
    out_ref[...] = lax.dot_general(
        head_ref[...], emb_ref[...],
        dimension_numbers=(((1,), (1,)), ((), ())),
        preferred_element_type=jnp.float32,
    )


def _make_table(head_padded, emb):
    return pl.pallas_call(
        _table_body,
        out_shape=jax.ShapeDtypeStruct((VPAD, VOCAB), jnp.float32),
    )(head_padded, emb)


def _gather_body(table_hbm, idst_hbm, out_hbm, dump_hbm,
                 ids_v, tbl_v, tile0, tile1, sem0, sem1):
    wid = lax.axis_index("s") * NC + lax.axis_index("c")
    pltpu.sync_copy(idst_hbm, ids_v)

    tiles = (tile0, tile1)
    sems = (sem0, sem1)

    def _compute(l, tile):
        row_splat = [jnp.full((16,), r, jnp.int32) for r in range(VT)]

        @plsc.parallel_loop(0, B_GROUPS, 1, unroll=4)
        def per_group(g):
            goff = pl.multiple_of(g * 16, 16)
            idxg = ids_v[l, pl.ds(goff, 16)]
            xs = [plsc.load_gather(tbl_v, [row_splat[r], idxg])
                  for r in range(VT)]
            for r in range(VT):
                tile[r, pl.ds(goff, 16)] = xs[r]

    def _emit(vt, l, buf):
        v0 = pl.multiple_of(vt * VT, VT)

        @pl.when(vt <= LAST_FULL_VT)
        def _():
            pltpu.async_copy(tiles[buf], out_hbm.at[l, pl.ds(v0, VT), :],
                             sems[buf])

        @pl.when(vt == PART_VT)
        def _():
            pltpu.async_copy(tiles[buf].at[pl.ds(0, PART_ROWS), :],
                             out_hbm.at[l, pl.ds(v0, PART_ROWS), :],
                             sems[buf])
            pltpu.async_copy(tiles[buf].at[pl.ds(PART_ROWS, VT - PART_ROWS), :],
                             dump_hbm.at[pl.ds(0, VT - PART_ROWS), :],
                             sems[buf])

        @pl.when(vt > PART_VT)
        def _():
            pltpu.async_copy(tiles[buf], dump_hbm, sems[buf])

    def _drain(buf):
        pltpu.make_async_copy(out_hbm.at[0, pl.ds(0, VT), :], tiles[buf],
                              sems[buf]).wait()

    def unit(t, carry):
        vt = wid * TILES_PER_W + t // HIST
        l = t % HIST

        @pl.when(l == 0)
        def _():
            pltpu.sync_copy(
                table_hbm.at[pl.ds(pl.multiple_of(vt * VT, VT), VT), :],
                tbl_v)

        @pl.when(jnp.logical_and(t >= 2, t % 2 == 0))
        def _():
            _drain(0)

        @pl.when(jnp.logical_and(t >= 2, t % 2 == 1))
        def _():
            _drain(1)

        @pl.when(t % 2 == 0)
        def _():
            _compute(l, tiles[0])
            _emit(vt, l, 0)

        @pl.when(t % 2 == 1)
        def _():
            _compute(l, tiles[1])
            _emit(vt, l, 1)

        return carry

    lax.fori_loop(0, UNITS, unit, 0)
    _drain(0)
    _drain(1)


def _gather_cols(table, ids_t):
    mesh = plsc.VectorSubcoreMesh(core_axis_name="c", subcore_axis_name="s")
    k = pl.kernel(
        _gather_body,
        out_type=(jax.ShapeDtypeStruct((HIST, VOCAB, BATCH), jnp.float32),
                  jax.ShapeDtypeStruct((VT, BATCH), jnp.float32)),
        mesh=mesh,
        compiler_params=pltpu.CompilerParams(needs_layout_passes=False),
        scratch_types=[
            pltpu.VMEM((HIST, BATCH), jnp.int32),
            pltpu.VMEM((VT, VOCAB), jnp.float32),
            pltpu.VMEM((VT, BATCH), jnp.float32),
            pltpu.VMEM((VT, BATCH), jnp.float32),
            pltpu.SemaphoreType.DMA,
            pltpu.SemaphoreType.DMA,
        ],
    )
    out, _ = k(table, ids_t)
    return out


def kernel(ids, emb_weight, head_weight):
    head_padded = jnp.pad(head_weight, ((0, VPAD - VOCAB), (0, 0)))
    table = _make_table(head_padded, emb_weight)
    ids_t = ids.astype(jnp.int32).T
    out_lvb = _gather_cols(table, ids_t)
    return jnp.transpose(out_lvb, (2, 0, 1))


# speedup vs baseline: 1.5681x; 1.5681x over previous
import functools

import jax
import jax.numpy as jnp
from jax import lax
from jax.experimental import pallas as pl
from jax.experimental.pallas import tpu as pltpu
from jax.experimental.pallas import tpu_sc as plsc

VOCAB = 1000
VPAD = 1024
DIM = 128
BATCH = 1024
HIST = 50
VT = 16
NC, NS = 2, 16
NW = NC * NS
TILES_PER_W = (VPAD // VT) // NW
B_GROUPS = BATCH // 16
UNITS = TILES_PER_W * HIST
LAST_FULL_VT = VOCAB // VT - 1
PART_VT = VOCAB // VT
PART_ROWS = VOCAB % VT


def _table_body(head_ref, emb_ref, out_ref):
    out_ref[...] = lax.dot_general(
        head_ref[...], emb_ref[...],
        dimension_numbers=(((1,), (1,)), ((), ())),
        preferred_element_type=jnp.float32,
    )


def _make_table(head_padded, emb):
    return pl.pallas_call(
        _table_body,
        out_shape=jax.ShapeDtypeStruct((VPAD, VOCAB), jnp.float32),
    )(head_padded, emb)


def _gather_body(table_hbm, idst_hbm, out_hbm, dump_hbm,
                 ids_v, tbl_v, tile0, tile1, sem0, sem1):
    wid = lax.axis_index("s") * NC + lax.axis_index("c")
    pltpu.sync_copy(idst_hbm, ids_v)

    tiles = (tile0, tile1)
    sems = (sem0, sem1)

    def _compute(l, tile):
        @plsc.parallel_loop(0, B_GROUPS, 1, unroll=4)
        def per_group(g):
            goff = pl.multiple_of(g * 16, 16)
            idxg = ids_v[l, pl.ds(goff, 16)]
            xs = [plsc.load_gather(tbl_v.at[pl.ds(r * VOCAB, VOCAB)], [idxg])
                  for r in range(VT)]
            for r in range(VT):
                tile[r, pl.ds(goff, 16)] = xs[r]

    def _emit(vt, l, buf):
        v0 = pl.multiple_of(vt * VT, VT)

        @pl.when(vt <= LAST_FULL_VT)
        def _():
            pltpu.async_copy(tiles[buf], out_hbm.at[l, pl.ds(v0, VT), :],
                             sems[buf])

        @pl.when(vt == PART_VT)
        def _():
            pltpu.async_copy(tiles[buf].at[pl.ds(0, PART_ROWS), :],
                             out_hbm.at[l, pl.ds(v0, PART_ROWS), :],
                             sems[buf])
            pltpu.async_copy(tiles[buf].at[pl.ds(PART_ROWS, VT - PART_ROWS), :],
                             dump_hbm.at[pl.ds(0, VT - PART_ROWS), :],
                             sems[buf])

        @pl.when(vt > PART_VT)
        def _():
            pltpu.async_copy(tiles[buf], dump_hbm, sems[buf])

    def _drain(buf):
        pltpu.make_async_copy(out_hbm.at[0, pl.ds(0, VT), :], tiles[buf],
                              sems[buf]).wait()

    def unit(t, carry):
        vt = wid * TILES_PER_W + t // HIST
        l = t % HIST

        @pl.when(l == 0)
        def _():
            pltpu.sync_copy(
                table_hbm.at[pl.ds(pl.multiple_of(vt * (VT * VOCAB), 8),
                                   VT * VOCAB)],
                tbl_v)

        @pl.when(jnp.logical_and(t >= 2, t % 2 == 0))
        def _():
            _drain(0)

        @pl.when(jnp.logical_and(t >= 2, t % 2 == 1))
        def _():
            _drain(1)

        @pl.when(t % 2 == 0)
        def _():
            _compute(l, tiles[0])
            _emit(vt, l, 0)

        @pl.when(t % 2 == 1)
        def _():
            _compute(l, tiles[1])
            _emit(vt, l, 1)

        return carry

    lax.fori_loop(0, UNITS, unit, 0)
    _drain(0)
    _drain(1)


def _gather_cols(table, ids_t):
    mesh = plsc.VectorSubcoreMesh(core_axis_name="c", subcore_axis_name="s")
    k = pl.kernel(
        _gather_body,
        out_type=(jax.ShapeDtypeStruct((HIST, VOCAB, BATCH), jnp.float32),
                  jax.ShapeDtypeStruct((VT, BATCH), jnp.float32)),
        mesh=mesh,
        compiler_params=pltpu.CompilerParams(needs_layout_passes=False),
        scratch_types=[
            pltpu.VMEM((HIST, BATCH), jnp.int32),
            pltpu.VMEM((VT * VOCAB,), jnp.float32),
            pltpu.VMEM((VT, BATCH), jnp.float32),
            pltpu.VMEM((VT, BATCH), jnp.float32),
            pltpu.SemaphoreType.DMA,
            pltpu.SemaphoreType.DMA,
        ],
    )
    out, _ = k(table, ids_t)
    return out


def kernel(ids, emb_weight, head_weight):
    head_padded = jnp.pad(head_weight, ((0, VPAD - VOCAB), (0, 0)))
    table = _make_table(head_padded, emb_weight).reshape(-1)
    ids_t = ids.astype(jnp.int32).T
    out_lvb = _gather_cols(table, ids_t)
    return jnp.transpose(out_lvb, (2, 0, 1))


# speedup vs baseline: 1.6031x; 1.0224x over previous
import functools

import jax
import jax.numpy as jnp
from jax import lax
from jax.experimental import pallas as pl
from jax.experimental.pallas import tpu as pltpu
from jax.experimental.pallas import tpu_sc as plsc

VOCAB = 1000
VPAD = 1024
DIM = 128
BATCH = 1024
HIST = 50
VT = 16
NC, NS = 2, 16
NW = NC * NS
TILES_PER_W = (VPAD // VT) // NW
B_GROUPS = BATCH // 16
UNITS = TILES_PER_W * HIST
LAST_FULL_VT = VOCAB // VT - 1
PART_VT = VOCAB // VT
PART_ROWS = VOCAB % VT


def _table_body(head_ref, emb_ref, out_ref):
    out_ref[pl.ds(0, VOCAB), :] = lax.dot_general(
        head_ref[...], emb_ref[...],
        dimension_numbers=(((1,), (1,)), ((), ())),
        preferred_element_type=jnp.float32,
    )


def _make_table(head, emb):
    return pl.pallas_call(
        _table_body,
        out_shape=jax.ShapeDtypeStruct((VPAD, VOCAB), jnp.float32),
    )(head, emb)


def _gather_body(table_hbm, idst_hbm, out_hbm, dump_hbm,
                 ids_v, tbl_v, tile0, tile1, sem0, sem1, sem_tbl):
    wid = lax.axis_index("s") * NC + lax.axis_index("c")
    pltpu.sync_copy(idst_hbm, ids_v)

    tiles = (tile0, tile1)
    sems = (sem0, sem1)

    def _compute(l, tile):
        @plsc.parallel_loop(0, B_GROUPS, 1, unroll=2)
        def per_group(g):
            goff = pl.multiple_of(g * 16, 16)
            idxg = ids_v[l, pl.ds(goff, 16)]
            xs = [plsc.load_gather(tbl_v.at[pl.ds(r * VOCAB, VOCAB)], [idxg])
                  for r in range(VT)]
            for r in range(VT):
                tile[r, pl.ds(goff, 16)] = xs[r]

    def _emit(vt, l, buf):
        v0 = pl.multiple_of(vt * VT, VT)

        @pl.when(vt <= LAST_FULL_VT)
        def _():
            pltpu.async_copy(tiles[buf], out_hbm.at[l, pl.ds(v0, VT), :],
                             sems[buf])

        @pl.when(vt == PART_VT)
        def _():
            pltpu.async_copy(tiles[buf].at[pl.ds(0, PART_ROWS), :],
                             out_hbm.at[l, pl.ds(v0, PART_ROWS), :],
                             sems[buf])
            pltpu.async_copy(tiles[buf].at[pl.ds(PART_ROWS, VT - PART_ROWS), :],
                             dump_hbm.at[pl.ds(0, VT - PART_ROWS), :],
                             sems[buf])

        @pl.when(vt > PART_VT)
        def _():
            pltpu.async_copy(tiles[buf], dump_hbm, sems[buf])

    def _drain(buf):
        pltpu.make_async_copy(out_hbm.at[0, pl.ds(0, VT), :], tiles[buf],
                              sems[buf]).wait()

    def unit(t, carry):
        vt = wid * TILES_PER_W + t // HIST
        l = t % HIST

        @pl.when(l == 0)
        def _():
            pltpu.sync_copy(
                table_hbm.at[pl.ds(pl.multiple_of(vt * (VT * VOCAB), 8),
                                   VT * VOCAB)],
                tbl_v)

        @pl.when(jnp.logical_and(t >= 2, t % 2 == 0))
        def _():
            _drain(0)

        @pl.when(jnp.logical_and(t >= 2, t % 2 == 1))
        def _():
            _drain(1)

        @pl.when(t % 2 == 0)
        def _():
            _compute(l, tiles[0])
            _emit(vt, l, 0)

        @pl.when(t % 2 == 1)
        def _():
            _compute(l, tiles[1])
            _emit(vt, l, 1)

        return carry

    lax.fori_loop(0, UNITS, unit, 0)
    _drain(0)
    _drain(1)


def _gather_cols(table, ids_t):
    mesh = plsc.VectorSubcoreMesh(core_axis_name="c", subcore_axis_name="s")
    k = pl.kernel(
        _gather_body,
        out_type=(jax.ShapeDtypeStruct((HIST, VOCAB, BATCH), jnp.float32),
                  jax.ShapeDtypeStruct((VT, BATCH), jnp.float32)),
        mesh=mesh,
        compiler_params=pltpu.CompilerParams(needs_layout_passes=False),
        scratch_types=[
            pltpu.VMEM((HIST, BATCH), jnp.int32),
            pltpu.VMEM((VT * VOCAB,), jnp.float32),
            pltpu.VMEM((VT, BATCH), jnp.float32),
            pltpu.VMEM((VT, BATCH), jnp.float32),
            pltpu.SemaphoreType.DMA,
            pltpu.SemaphoreType.DMA,
            pltpu.SemaphoreType.DMA,
        ],
    )
    out, _ = k(table, ids_t)
    return out


def kernel(ids, emb_weight, head_weight):
    table = _make_table(head_weight, emb_weight).reshape(-1)
    ids_t = ids.astype(jnp.int32).T
    out_lvb = _gather_cols(table, ids_t)
    return jnp.transpose(out_lvb, (2, 0, 1))


# speedup vs baseline: 1.6140x; 1.0068x over previous
import functools

import jax
import jax.numpy as jnp
from jax import lax
from jax.experimental import pallas as pl
from jax.experimental.pallas import tpu as pltpu
from jax.experimental.pallas import tpu_sc as plsc

VOCAB = 1000
VPAD = 1024
DIM = 128
BATCH = 1024
HIST = 50
VT = 16
NC, NS = 2, 16
NW = NC * NS
TILES_PER_W = (VPAD // VT) // NW
B_GROUPS = BATCH // 16
UNITS = TILES_PER_W * HIST
LAST_FULL_VT = VOCAB // VT - 1
PART_VT = VOCAB // VT
PART_ROWS = VOCAB % VT


def _table_body(head_ref, emb_ref, out_ref):
    out_ref[pl.ds(0, VOCAB), :] = lax.dot_general(
        head_ref[...], emb_ref[...],
        dimension_numbers=(((1,), (1,)), ((), ())),
        preferred_element_type=jnp.float32,
    )


def _make_table(head, emb):
    return pl.pallas_call(
        _table_body,
        out_shape=jax.ShapeDtypeStruct((VPAD, VOCAB), jnp.float32),
    )(head, emb)


def _gather_body(table_hbm, idst_hbm, out_hbm, dump_hbm,
                 ids_v, tbl_v, tile0, tile1, tile2, sem0, sem1, sem2,
                 sem_tbl):
    wid = lax.axis_index("s") * NC + lax.axis_index("c")
    ids_cp = pltpu.async_copy(idst_hbm, ids_v, sem_tbl)

    tiles = (tile0, tile1, tile2)
    sems = (sem0, sem1, sem2)
    NBUF = 3

    def _compute(l, tile):
        @plsc.parallel_loop(0, B_GROUPS, 1, unroll=2)
        def per_group(g):
            goff = pl.multiple_of(g * 16, 16)
            idxg = ids_v[l, pl.ds(goff, 16)]
            xs = [plsc.load_gather(tbl_v.at[pl.ds(r * VOCAB, VOCAB)], [idxg])
                  for r in range(VT)]
            for r in range(VT):
                tile[r, pl.ds(goff, 16)] = xs[r]

    def _emit(vt, l, buf):
        v0 = pl.multiple_of(vt * VT, VT)

        @pl.when(vt <= LAST_FULL_VT)
        def _():
            pltpu.async_copy(tiles[buf], out_hbm.at[l, pl.ds(v0, VT), :],
                             sems[buf])

        @pl.when(vt == PART_VT)
        def _():
            pltpu.async_copy(tiles[buf].at[pl.ds(0, PART_ROWS), :],
                             out_hbm.at[l, pl.ds(v0, PART_ROWS), :],
                             sems[buf])
            pltpu.async_copy(tiles[buf].at[pl.ds(PART_ROWS, VT - PART_ROWS), :],
                             dump_hbm.at[pl.ds(0, VT - PART_ROWS), :],
                             sems[buf])

        @pl.when(vt > PART_VT)
        def _():
            pltpu.async_copy(tiles[buf], dump_hbm, sems[buf])

    def _drain(buf):
        pltpu.make_async_copy(out_hbm.at[0, pl.ds(0, VT), :], tiles[buf],
                              sems[buf]).wait()

    def unit(t, carry):
        vt = wid * TILES_PER_W + t // HIST
        l = t % HIST

        @pl.when(l == 0)
        def _():
            pltpu.sync_copy(
                table_hbm.at[pl.ds(pl.multiple_of(vt * (VT * VOCAB), 8),
                                   VT * VOCAB)],
                tbl_v)

        for b in range(NBUF):
            @pl.when(jnp.logical_and(t >= NBUF, t % NBUF == b))
            def _(b=b):
                _drain(b)

        for b in range(NBUF):
            @pl.when(t % NBUF == b)
            def _(b=b):
                _compute(l, tiles[b])
                _emit(vt, l, b)

        return carry

    ids_cp.wait()
    lax.fori_loop(0, UNITS, unit, 0)
    for b in range(NBUF):
        _drain(b)


def _gather_cols(table, ids_t):
    mesh = plsc.VectorSubcoreMesh(core_axis_name="c", subcore_axis_name="s")
    k = pl.kernel(
        _gather_body,
        out_type=(jax.ShapeDtypeStruct((HIST, VOCAB, BATCH), jnp.float32),
                  jax.ShapeDtypeStruct((VT, BATCH), jnp.float32)),
        mesh=mesh,
        compiler_params=pltpu.CompilerParams(needs_layout_passes=False),
        scratch_types=[
            pltpu.VMEM((HIST, BATCH), jnp.int32),
            pltpu.VMEM((VT * VOCAB,), jnp.float32),
            pltpu.VMEM((VT, BATCH), jnp.float32),
            pltpu.VMEM((VT, BATCH), jnp.float32),
            pltpu.VMEM((VT, BATCH), jnp.float32),
            pltpu.SemaphoreType.DMA,
            pltpu.SemaphoreType.DMA,
            pltpu.SemaphoreType.DMA,
            pltpu.SemaphoreType.DMA,
        ],
    )
    out, _ = k(table, ids_t)
    return out


def kernel(ids, emb_weight, head_weight):
    table = _make_table(head_weight, emb_weight).reshape(-1)
    ids_t = ids.astype(jnp.int32).T
    out_lvb = _gather_cols(table, ids_t)
    return jnp.transpose(out_lvb, (2, 0, 1))
